# baseline (speedup 1.0000x reference)
import functools

import jax
import jax.numpy as jnp
from jax import lax
from jax.experimental import pallas as pl
from jax.experimental.pallas import tpu as pltpu
from jax.experimental.pallas import tpu_sc as plsc

N = 10000
D = 128
H = D // 2
G = 64
A = 8

NC = 2
NS = 16
C = 128
NCH = 160
EPS = C * NCH
EP = NS * EPS
N_ACC = 10240
ZROWS = N_ACC // NS
CW = 16

RB = 1000
NB = N // RB
HIGHEST = lax.Precision.HIGHEST


def _make_agg(with_cnt):
  mesh = plsc.VectorSubcoreMesh(core_axis_name="c", subcore_axis_name="s")
  out_type = [jax.ShapeDtypeStruct((NC, N_ACC, H), jnp.float32)]
  R = 4
  DR = R // 2
  scratch = [
      pltpu.VMEM((NCH, C), jnp.int32),
      pltpu.VMEM((NCH, C), jnp.int32),
      pltpu.VMEM((R, C, H), jnp.float32),
      pltpu.VMEM_SHARED((N_ACC, H), jnp.float32),
      [pltpu.SemaphoreType.DMA] * R,
      [pltpu.SemaphoreType.DMA] * R,
  ]
  if with_cnt:
    out_type.append(jax.ShapeDtypeStruct((NS, N_ACC), jnp.float32))
    scratch += [
        pltpu.VMEM((N_ACC,), jnp.float32),
    ]

  def body(h2_hbm, src_hbm, dst_hbm, zf_hbm, *rest):
    if with_cnt:
      (zc_hbm, out_hbm, cnt_hbm,
       src_v, dst_v, ring, acc, gsem, ssem, cnt_t) = rest
    else:
      (out_hbm, src_v, dst_v, ring, acc, gsem, ssem) = rest
    c = lax.axis_index("c")
    s = lax.axis_index("s")

    pltpu.sync_copy(src_hbm.at[c, s], src_v)
    pltpu.sync_copy(dst_hbm.at[s], dst_v)
    pltpu.sync_copy(zf_hbm, acc.at[pl.ds(s * ZROWS, ZROWS)])
    if with_cnt:
      pltpu.sync_copy(zc_hbm, cnt_t)
    plsc.subcore_barrier()

    def gather_start(slot, j):
      pltpu.async_copy(h2_hbm.at[src_v.at[j]], ring.at[slot], gsem[slot])

    def gather_wait(slot, j):
      pltpu.make_async_copy(h2_hbm.at[src_v.at[j]], ring.at[slot],
                            gsem[slot]).wait()

    def scatter_start(slot, j):
      pltpu.async_copy(ring.at[slot], acc.at[dst_v.at[j]], ssem[slot],
                       add=True)

    def scatter_wait(slot, j):
      pltpu.make_async_copy(ring.at[slot], acc.at[dst_v.at[j]],
                            ssem[slot]).wait()

    ones16 = jnp.ones((16,), jnp.float32)

    for b in range(R):
      gather_start(b, b)

    @pl.loop(0, NCH, step=R)
    def _(j):
      for b in range(R):
        jj = j + b
        gather_wait(b, jj)
        scatter_start(b, jj)
        if with_cnt:
          for k in range(C // 16):
            plsc.addupdate_scatter(
                cnt_t, [dst_v[jj, pl.ds(k * 16, 16)]], ones16)
        pj = jj - DR
        q = (b + DR) % R

        @pl.when(pj >= 0)
        def _():
          scatter_wait(q, pj)

          @pl.when(pj + R < NCH)
          def _():
            gather_start(q, pj + R)

    for t in range(NCH - DR, NCH):
      scatter_wait(t % R, t)
    plsc.subcore_barrier()
    pltpu.sync_copy(acc.at[pl.ds(s * ZROWS, ZROWS)],
                    out_hbm.at[c, pl.ds(s * ZROWS, ZROWS)])
    if with_cnt:

      @pl.when(c == 0)
      def _():
        pltpu.sync_copy(cnt_t, cnt_hbm.at[s])

  return pl.kernel(body, out_type=out_type, mesh=mesh, scratch_types=scratch,
                   compiler_params=pltpu.CompilerParams(
                       use_tc_tiling_on_sc=False,
                       needs_layout_passes=False))


_agg_cnt = _make_agg(True)
_agg = _make_agg(False)


def _mm(a, b):
  return jnp.dot(a, b, preferred_element_type=jnp.float32)


def _dense_body(ap, cnt, hp, wlr, bl, o):
  cv = jnp.maximum(jnp.sum(cnt[...][0], axis=0), 1.0)[:, None]
  a = ap[...] / cv[None]
  z = jnp.concatenate([a[0], a[1], hp[...]], axis=1)
  o[...] = jnp.maximum(_mm(z, wlr[...]) + bl[...], 0.0)


_dense = pl.pallas_call(
    _dense_body,
    grid=(NB,),
    in_specs=[
        pl.BlockSpec((NC, RB, H), lambda i: (0, i, 0)),
        pl.BlockSpec((1, NS, RB), lambda i: (i, 0, 0)),
        pl.BlockSpec((RB, D), lambda i: (i, 0)),
        pl.BlockSpec((2 * D, D), lambda i: (0, 0)),
        pl.BlockSpec((1, D), lambda i: (0, 0)),
    ],
    out_specs=pl.BlockSpec((RB, D), lambda i: (i, 0)),
    out_shape=jax.ShapeDtypeStruct((N, D), jnp.float32),
)


def _final_body(ap, cnt, hp, bt, wlr, bl, wlin, blin, o, s_acc):
  i = pl.program_id(0)

  @pl.when(i == 0)
  def _():
    s_acc[...] = jnp.zeros_like(s_acc)

  cv = jnp.maximum(jnp.sum(cnt[...][0], axis=0), 1.0)[:, None]
  a = ap[...] / cv[None]
  z = jnp.concatenate([a[0], a[1], hp[...]], axis=1)
  h3 = jnp.maximum(_mm(z, wlr[...]) + bl[...], 0.0)
  p = jnp.tanh(_mm(h3, wlin[...]) + blin[...])
  onehot = (bt[...] == lax.broadcasted_iota(jnp.int32, (RB, G), 1)
            ).astype(jnp.float32)
  dn = (((0,), (0,)), ((), ()))
  pc = jnp.concatenate([p, jnp.ones((RB, 1), jnp.float32)], axis=1)
  s_acc[...] += lax.dot_general(onehot, pc, dn,
                                preferred_element_type=jnp.float32)

  @pl.when(i == NB - 1)
  def _():
    o[...] = s_acc[:, :A] / jnp.maximum(s_acc[:, A:], 1.0)


_final = pl.pallas_call(
    _final_body,
    grid=(NB,),
    in_specs=[
        pl.BlockSpec((NC, RB, H), lambda i: (0, i, 0)),
        pl.BlockSpec((1, NS, RB), lambda i: (i, 0, 0)),
        pl.BlockSpec((RB, D), lambda i: (i, 0)),
        pl.BlockSpec((RB, 1), lambda i: (i, 0)),
        pl.BlockSpec((2 * D, D), lambda i: (0, 0)),
        pl.BlockSpec((1, D), lambda i: (0, 0)),
        pl.BlockSpec((D, A), lambda i: (0, 0)),
        pl.BlockSpec((1, A), lambda i: (0, 0)),
    ],
    out_specs=pl.BlockSpec((G, A), lambda i: (0, 0)),
    out_shape=jax.ShapeDtypeStruct((G, A), jnp.float32),
    scratch_shapes=[
        pltpu.VMEM((G, A + 1), jnp.float32),
    ],
)


def _split_w(w):
  wt = w.T
  return jnp.stack([wt[:H], wt[H:]])


def kernel(x, edge_index, batch, W1l, b1l, W1r, W2l, b2l, W2r,
           W3l, b3l, W3r, Wlin, blin):
  src = edge_index[0]
  dst = edge_index[1]
  pad = EP - src.shape[0]
  src_p = jnp.concatenate([src, jnp.zeros((pad,), jnp.int32)])
  srcr = jnp.stack([2 * src_p, 2 * src_p + 1]).reshape(NC, NS, NCH, C)
  dstr = jnp.concatenate([dst, jnp.full((pad,), N, jnp.int32)]).reshape(
      NS, NCH, C)
  zf = jnp.zeros((ZROWS, H), jnp.float32)
  zc = jnp.zeros((N_ACC,), jnp.float32)
  bt = batch.reshape(N, 1)

  w1 = jnp.concatenate([W1l.T, W1r.T], axis=0)
  w2 = jnp.concatenate([W2l.T, W2r.T], axis=0)
  w3 = jnp.concatenate([W3l.T, W3r.T], axis=0)

  a1p, cp = _agg_cnt(x.reshape(NC * N, H), srcr, dstr, zf, zc)
  cnt = cp[:, :N].reshape(NS, NB, RB).transpose(1, 0, 2)
  h1 = _dense(a1p, cnt, x, w1, b1l.reshape(1, D))
  a2p, = _agg(h1.reshape(NC * N, H), srcr, dstr, zf)
  h2 = _dense(a2p, cnt, h1, w2, b2l.reshape(1, D))
  a3p, = _agg(h2.reshape(NC * N, H), srcr, dstr, zf)
  return _final(a3p, cnt, h2, bt, w3, b3l.reshape(1, D),
                Wlin.T, blin.reshape(1, A))


# speedup vs baseline: 1.1531x; 1.1531x over previous
import functools

import jax
import jax.numpy as jnp
from jax import lax
from jax.experimental import pallas as pl
from jax.experimental.pallas import tpu as pltpu
from jax.experimental.pallas import tpu_sc as plsc

N = 10000
D = 128
H = D // 2
G = 64
A = 8

NC = 2
NS = 16
C = 128
NCH = 160
EPS = C * NCH
EP = NS * EPS
N_ACC = 10240
ZROWS = N_ACC // NS
CW = 16

RB = 1000
NB = N // RB
HIGHEST = lax.Precision.HIGHEST


def _make_agg(with_cnt):
  mesh = plsc.VectorSubcoreMesh(core_axis_name="c", subcore_axis_name="s")
  out_type = [jax.ShapeDtypeStruct((NC, N_ACC, H), jnp.float32)]
  R = 4
  DR = R // 2
  scratch = [
      pltpu.VMEM((NCH, C), jnp.int32),
      pltpu.VMEM((NCH, C), jnp.int32),
      pltpu.VMEM((R, C, H), jnp.float32),
      pltpu.VMEM_SHARED((N_ACC, H), jnp.float32),
      [pltpu.SemaphoreType.DMA] * R,
      [pltpu.SemaphoreType.DMA] * R,
  ]
  if with_cnt:
    out_type.append(jax.ShapeDtypeStruct((NS, N_ACC), jnp.float32))
    scratch += [
        pltpu.VMEM((N_ACC,), jnp.float32),
    ]

  def body(h2_hbm, src_hbm, dst_hbm, zf_hbm, *rest):
    if with_cnt:
      (zc_hbm, out_hbm, cnt_hbm,
       src_v, dst_v, ring, acc, gsem, ssem, cnt_t) = rest
    else:
      (out_hbm, src_v, dst_v, ring, acc, gsem, ssem) = rest
    c = lax.axis_index("c")
    s = lax.axis_index("s")

    pltpu.sync_copy(src_hbm.at[c, s], src_v)
    pltpu.sync_copy(dst_hbm.at[s], dst_v)
    pltpu.sync_copy(zf_hbm, acc.at[pl.ds(s * ZROWS, ZROWS)])
    if with_cnt:
      pltpu.sync_copy(zc_hbm, cnt_t)
    plsc.subcore_barrier()

    def gather_start(slot, j):
      pltpu.async_copy(h2_hbm.at[src_v.at[j]], ring.at[slot], gsem[slot])

    def gather_wait(slot, j):
      pltpu.make_async_copy(h2_hbm.at[src_v.at[j]], ring.at[slot],
                            gsem[slot]).wait()

    def scatter_start(slot, j):
      pltpu.async_copy(ring.at[slot], acc.at[dst_v.at[j]], ssem[slot],
                       add=True)

    def scatter_wait(slot, j):
      pltpu.make_async_copy(ring.at[slot], acc.at[dst_v.at[j]],
                            ssem[slot]).wait()

    ones16 = jnp.ones((16,), jnp.float32)

    for b in range(R):
      gather_start(b, b)

    @pl.loop(0, NCH, step=R)
    def _(j):
      for b in range(R):
        jj = j + b
        gather_wait(b, jj)
        scatter_start(b, jj)
        if with_cnt:
          for k in range(C // 16):
            plsc.addupdate_scatter(
                cnt_t, [dst_v[jj, pl.ds(k * 16, 16)]], ones16)
        pj = jj - DR
        q = (b + DR) % R

        @pl.when(pj >= 0)
        def _():
          scatter_wait(q, pj)

          @pl.when(pj + R < NCH)
          def _():
            gather_start(q, pj + R)

    for t in range(NCH - DR, NCH):
      scatter_wait(t % R, t)
    plsc.subcore_barrier()
    pltpu.sync_copy(acc.at[pl.ds(s * ZROWS, ZROWS)],
                    out_hbm.at[c, pl.ds(s * ZROWS, ZROWS)])
    if with_cnt:

      @pl.when(c == 0)
      def _():
        pltpu.sync_copy(cnt_t, cnt_hbm.at[s])

  return pl.kernel(body, out_type=out_type, mesh=mesh, scratch_types=scratch,
                   compiler_params=pltpu.CompilerParams(
                       use_tc_tiling_on_sc=False,
                       needs_layout_passes=False))


_agg_cnt = _make_agg(True)
_agg = _make_agg(False)


def _mm(a, b):
  return jnp.dot(a, b, preferred_element_type=jnp.float32)


def _dense_body(ap, cnt, hp, wlr, bl, o):
  cv = jnp.maximum(jnp.sum(cnt[...][0], axis=0), 1.0)[:, None]
  a = ap[...] / cv[None]
  z = jnp.concatenate([a[0], a[1], hp[...][0], hp[...][1]], axis=1)
  y = jnp.maximum(_mm(z, wlr[...]) + bl[...], 0.0)
  o[0] = y[:, :H]
  o[1] = y[:, H:]


_dense = pl.pallas_call(
    _dense_body,
    grid=(NB,),
    in_specs=[
        pl.BlockSpec((NC, RB, H), lambda i: (0, i, 0)),
        pl.BlockSpec((1, NS, RB), lambda i: (i, 0, 0)),
        pl.BlockSpec((NC, RB, H), lambda i: (0, i, 0)),
        pl.BlockSpec((2 * D, D), lambda i: (0, 0)),
        pl.BlockSpec((1, D), lambda i: (0, 0)),
    ],
    out_specs=pl.BlockSpec((NC, RB, H), lambda i: (0, i, 0)),
    out_shape=jax.ShapeDtypeStruct((NC, N, H), jnp.float32),
)


def _final_body(ap, cnt, hp, bt, wlr, bl, wlin, blin, o, s_acc):
  i = pl.program_id(0)

  @pl.when(i == 0)
  def _():
    s_acc[...] = jnp.zeros_like(s_acc)

  cv = jnp.maximum(jnp.sum(cnt[...][0], axis=0), 1.0)[:, None]
  a = ap[...] / cv[None]
  z = jnp.concatenate([a[0], a[1], hp[...][0], hp[...][1]], axis=1)
  h3 = jnp.maximum(_mm(z, wlr[...]) + bl[...], 0.0)
  p = jnp.tanh(_mm(h3, wlin[...]) + blin[...])
  onehot = (bt[...] == lax.broadcasted_iota(jnp.int32, (RB, G), 1)
            ).astype(jnp.float32)
  dn = (((0,), (0,)), ((), ()))
  pc = jnp.concatenate([p, jnp.ones((RB, 1), jnp.float32)], axis=1)
  s_acc[...] += lax.dot_general(onehot, pc, dn,
                                preferred_element_type=jnp.float32)

  @pl.when(i == NB - 1)
  def _():
    o[...] = s_acc[:, :A] / jnp.maximum(s_acc[:, A:], 1.0)


_final = pl.pallas_call(
    _final_body,
    grid=(NB,),
    in_specs=[
        pl.BlockSpec((NC, RB, H), lambda i: (0, i, 0)),
        pl.BlockSpec((1, NS, RB), lambda i: (i, 0, 0)),
        pl.BlockSpec((NC, RB, H), lambda i: (0, i, 0)),
        pl.BlockSpec((RB, 1), lambda i: (i, 0)),
        pl.BlockSpec((2 * D, D), lambda i: (0, 0)),
        pl.BlockSpec((1, D), lambda i: (0, 0)),
        pl.BlockSpec((D, A), lambda i: (0, 0)),
        pl.BlockSpec((1, A), lambda i: (0, 0)),
    ],
    out_specs=pl.BlockSpec((G, A), lambda i: (0, 0)),
    out_shape=jax.ShapeDtypeStruct((G, A), jnp.float32),
    scratch_shapes=[
        pltpu.VMEM((G, A + 1), jnp.float32),
    ],
)


def _split_w(w):
  wt = w.T
  return jnp.stack([wt[:H], wt[H:]])


def kernel(x, edge_index, batch, W1l, b1l, W1r, W2l, b2l, W2r,
           W3l, b3l, W3r, Wlin, blin):
  src = edge_index[0]
  dst = edge_index[1]
  pad = EP - src.shape[0]
  src_p = jnp.concatenate([src, jnp.zeros((pad,), jnp.int32)])
  srcr = jnp.stack([src_p, src_p + N]).reshape(NC, NS, NCH, C)
  dstr = jnp.concatenate([dst, jnp.full((pad,), N, jnp.int32)]).reshape(
      NS, NCH, C)
  zf = jnp.zeros((ZROWS, H), jnp.float32)
  zc = jnp.zeros((N_ACC,), jnp.float32)
  bt = batch.reshape(N, 1)

  xs = jnp.stack([x[:, :H], x[:, H:]])

  w1 = jnp.concatenate([W1l.T, W1r.T], axis=0)
  w2 = jnp.concatenate([W2l.T, W2r.T], axis=0)
  w3 = jnp.concatenate([W3l.T, W3r.T], axis=0)

  a1p, cp = _agg_cnt(xs.reshape(NC * N, H), srcr, dstr, zf, zc)
  cnt = cp[:, :N].reshape(NS, NB, RB).transpose(1, 0, 2)
  h1 = _dense(a1p, cnt, xs, w1, b1l.reshape(1, D))
  a2p, = _agg(h1.reshape(NC * N, H), srcr, dstr, zf)
  h2 = _dense(a2p, cnt, h1, w2, b2l.reshape(1, D))
  a3p, = _agg(h2.reshape(NC * N, H), srcr, dstr, zf)
  return _final(a3p, cnt, h2, bt, w3, b3l.reshape(1, D),
                Wlin.T, blin.reshape(1, A))


# speedup vs baseline: 1.1546x; 1.0012x over previous
import jax
import jax.numpy as jnp
from jax import lax
from jax.experimental import pallas as pl
from jax.experimental.pallas import tpu as pltpu
from jax.experimental.pallas import tpu_sc as plsc

N = 10000
D = 128
H = D // 2
G = 64
A = 8

NC = 2
NS = 16
C = 128
NCH = 160
EPS = C * NCH
EP = NS * EPS
N_ACC = 10240
ZROWS = N_ACC // NS

RB = 1000
NB = N // RB


def _make_agg(with_cnt):
  mesh = plsc.VectorSubcoreMesh(core_axis_name="c", subcore_axis_name="s")
  out_type = [jax.ShapeDtypeStruct((NC, N_ACC, H), jnp.float32)]
  R = 4
  DR = R // 2
  scratch = [
      pltpu.VMEM((NCH, C), jnp.int32),
      pltpu.VMEM((NCH, C), jnp.int32),
      pltpu.VMEM((R, C, H), jnp.float32),
      pltpu.VMEM_SHARED((N_ACC, H), jnp.float32),
      [pltpu.SemaphoreType.DMA] * R,
      [pltpu.SemaphoreType.DMA] * R,
  ]
  if with_cnt:
    out_type.append(jax.ShapeDtypeStruct((NS, N_ACC), jnp.float32))
    scratch += [
        pltpu.VMEM((N_ACC,), jnp.float32),
    ]

  def body(h2_hbm, src_hbm, dst_hbm, zf_hbm, *rest):
    if with_cnt:
      (zc_hbm, out_hbm, cnt_hbm,
       src_v, dst_v, ring, acc, gsem, ssem, cnt_t) = rest
    else:
      (out_hbm, src_v, dst_v, ring, acc, gsem, ssem) = rest
    c = lax.axis_index("c")
    s = lax.axis_index("s")

    pltpu.sync_copy(src_hbm.at[c, s], src_v)
    pltpu.sync_copy(dst_hbm.at[s], dst_v)
    pltpu.sync_copy(zf_hbm, acc.at[pl.ds(s * ZROWS, ZROWS)])
    if with_cnt:
      pltpu.sync_copy(zc_hbm, cnt_t)
    plsc.subcore_barrier()

    def gather_start(slot, j):
      pltpu.async_copy(h2_hbm.at[src_v.at[j]], ring.at[slot], gsem[slot])

    def gather_wait(slot, j):
      pltpu.make_async_copy(h2_hbm.at[src_v.at[j]], ring.at[slot],
                            gsem[slot]).wait()

    def scatter_start(slot, j):
      pltpu.async_copy(ring.at[slot], acc.at[dst_v.at[j]], ssem[slot],
                       add=True)

    def scatter_wait(slot, j):
      pltpu.make_async_copy(ring.at[slot], acc.at[dst_v.at[j]],
                            ssem[slot]).wait()

    ones16 = jnp.ones((16,), jnp.float32)

    for b in range(R):
      gather_start(b, b)

    @pl.loop(0, NCH, step=R)
    def _(j):
      for b in range(R):
        jj = j + b
        gather_wait(b, jj)
        scatter_start(b, jj)
        if with_cnt:
          for k in range(C // 16):
            plsc.addupdate_scatter(
                cnt_t, [dst_v[jj, pl.ds(k * 16, 16)]], ones16)
        pj = jj - DR
        q = (b + DR) % R

        @pl.when(pj >= 0)
        def _():
          scatter_wait(q, pj)

          @pl.when(pj + R < NCH)
          def _():
            gather_start(q, pj + R)

    for t in range(NCH - DR, NCH):
      scatter_wait(t % R, t)
    plsc.subcore_barrier()
    pltpu.sync_copy(acc.at[pl.ds(s * ZROWS, ZROWS)],
                    out_hbm.at[c, pl.ds(s * ZROWS, ZROWS)])
    if with_cnt:

      @pl.when(c == 0)
      def _():
        pltpu.sync_copy(cnt_t, cnt_hbm.at[s])

  return pl.kernel(body, out_type=out_type, mesh=mesh, scratch_types=scratch,
                   compiler_params=pltpu.CompilerParams(
                       use_tc_tiling_on_sc=False,
                       needs_layout_passes=False))


_agg_cnt = _make_agg(True)
_agg = _make_agg(False)


def _mm(a, b):
  return jnp.dot(a, b, preferred_element_type=jnp.float32)


def _dense_body(ap, cnt, hp, wlr, bl, o):
  cv = jnp.maximum(jnp.sum(cnt[...][0], axis=0), 1.0)[:, None]
  a = ap[...] / cv[None]
  z = jnp.concatenate([a[0], a[1], hp[...][0], hp[...][1]], axis=1)
  y = jnp.maximum(_mm(z, wlr[...]) + bl[...], 0.0)
  o[0] = y[:, :H]
  o[1] = y[:, H:]


_dense = pl.pallas_call(
    _dense_body,
    grid=(NB,),
    in_specs=[
        pl.BlockSpec((NC, RB, H), lambda i: (0, i, 0)),
        pl.BlockSpec((1, NS, RB), lambda i: (i, 0, 0)),
        pl.BlockSpec((NC, RB, H), lambda i: (0, i, 0)),
        pl.BlockSpec((2 * D, D), lambda i: (0, 0)),
        pl.BlockSpec((1, D), lambda i: (0, 0)),
    ],
    out_specs=pl.BlockSpec((NC, RB, H), lambda i: (0, i, 0)),
    out_shape=jax.ShapeDtypeStruct((NC, N, H), jnp.float32),
)


def _final_body(ap, cnt, hp, bt, wlr, bl, wlin, blin, o, s_acc):
  i = pl.program_id(0)

  @pl.when(i == 0)
  def _():
    s_acc[...] = jnp.zeros_like(s_acc)

  cv = jnp.maximum(jnp.sum(cnt[...][0], axis=0), 1.0)[:, None]
  a = ap[...] / cv[None]
  z = jnp.concatenate([a[0], a[1], hp[...][0], hp[...][1]], axis=1)
  h3 = jnp.maximum(_mm(z, wlr[...]) + bl[...], 0.0)
  p = jnp.tanh(_mm(h3, wlin[...]) + blin[...])
  onehot = (bt[...] == lax.broadcasted_iota(jnp.int32, (RB, G), 1)
            ).astype(jnp.float32)
  dn = (((0,), (0,)), ((), ()))
  pc = jnp.concatenate([p, jnp.ones((RB, 1), jnp.float32)], axis=1)
  s_acc[...] += lax.dot_general(onehot, pc, dn,
                                preferred_element_type=jnp.float32)

  @pl.when(i == NB - 1)
  def _():
    o[...] = s_acc[:, :A] / jnp.maximum(s_acc[:, A:], 1.0)


_final = pl.pallas_call(
    _final_body,
    grid=(NB,),
    in_specs=[
        pl.BlockSpec((NC, RB, H), lambda i: (0, i, 0)),
        pl.BlockSpec((1, NS, RB), lambda i: (i, 0, 0)),
        pl.BlockSpec((NC, RB, H), lambda i: (0, i, 0)),
        pl.BlockSpec((RB, 1), lambda i: (i, 0)),
        pl.BlockSpec((2 * D, D), lambda i: (0, 0)),
        pl.BlockSpec((1, D), lambda i: (0, 0)),
        pl.BlockSpec((D, A), lambda i: (0, 0)),
        pl.BlockSpec((1, A), lambda i: (0, 0)),
    ],
    out_specs=pl.BlockSpec((G, A), lambda i: (0, 0)),
    out_shape=jax.ShapeDtypeStruct((G, A), jnp.float32),
    scratch_shapes=[
        pltpu.VMEM((G, A + 1), jnp.float32),
    ],
)


def kernel(x, edge_index, batch, W1l, b1l, W1r, W2l, b2l, W2r,
           W3l, b3l, W3r, Wlin, blin):
  src = edge_index[0]
  dst = edge_index[1]
  pad = EP - src.shape[0]
  src_p = jnp.concatenate([src, jnp.zeros((pad,), jnp.int32)])
  srcr = jnp.stack([src_p, src_p + N]).reshape(NC, NS, NCH, C)
  dstr = jnp.concatenate([dst, jnp.full((pad,), N, jnp.int32)]).reshape(
      NS, NCH, C)
  zf = jnp.zeros((ZROWS, H), jnp.float32)
  zc = jnp.zeros((N_ACC,), jnp.float32)
  bt = batch.reshape(N, 1)

  xs = jnp.stack([x[:, :H], x[:, H:]])

  w1 = jnp.concatenate([W1l.T, W1r.T], axis=0)
  w2 = jnp.concatenate([W2l.T, W2r.T], axis=0)
  w3 = jnp.concatenate([W3l.T, W3r.T], axis=0)

  a1p, cp = _agg_cnt(xs.reshape(NC * N, H), srcr, dstr, zf, zc)
  cnt = cp[:, :N].reshape(NS, NB, RB).transpose(1, 0, 2)
  h1 = _dense(a1p, cnt, xs, w1, b1l.reshape(1, D))
  a2p, = _agg(h1.reshape(NC * N, H), srcr, dstr, zf)
  h2 = _dense(a2p, cnt, h1, w2, b2l.reshape(1, D))
  a3p, = _agg(h2.reshape(NC * N, H), srcr, dstr, zf)
  return _final(a3p, cnt, h2, bt, w3, b3l.reshape(1, D),
                Wlin.T, blin.reshape(1, A))
